# baseline (speedup 1.0000x reference)
import functools
import jax
import jax.numpy as jnp
from jax import lax
from jax.experimental import pallas as pl
from jax.experimental.pallas import tpu as pltpu
from jax.experimental.pallas import tpu_sc as plsc

D_MODEL = 1024
SCALE = 32.0
NC, NS, L = 2, 16, 16
NW = NC * NS
CHUNK = 16
NBUF = 4
PD = 2


def _body(ids_hbm, table_hbm, out_hbm, *scratch):
    idx_v = scratch[0]
    bufs = scratch[1:1 + NBUF]
    gsems = scratch[1 + NBUF:1 + 2 * NBUF]
    ssems = scratch[1 + 2 * NBUF:1 + 3 * NBUF]
    R, S = ids_hbm.shape
    B = R * S
    bpw = B // NW
    n = bpw // CHUNK
    wid = lax.axis_index("s") * NC + lax.axis_index("c")
    base = wid * bpw
    r0 = base // S
    col0 = base % S

    pltpu.sync_copy(ids_hbm.at[r0, pl.ds(col0, bpw)], idx_v)

    def gather(c, b):
        return pltpu.make_async_copy(
            table_hbm.at[idx_v.at[pl.ds(c * CHUNK, CHUNK)]], bufs[b], gsems[b]
        )

    def store(c, b):
        return pltpu.make_async_copy(
            bufs[b], out_hbm.at[r0, pl.ds(col0 + c * CHUNK, CHUNK)], ssems[b]
        )

    for c0 in range(PD):
        gather(c0, c0).start()

    def scale_buf(buf):
        @plsc.parallel_loop(0, CHUNK, 1)
        def _(j):
            @plsc.parallel_loop(0, D_MODEL, L, unroll=4)
            def _(k):
                buf[j, pl.ds(k, L)] = buf[j, pl.ds(k, L)] * SCALE

    def outer(t, _):
        for b in range(NBUF):
            c = NBUF * t + b
            nb = (b + PD) % NBUF

            @pl.when(c >= NBUF - PD)
            def _():
                store(c + PD - NBUF, nb).wait()

            @pl.when(c + PD < n)
            def _():
                gather(c + PD, nb).start()

            gather(c, b).wait()
            scale_buf(bufs[b])
            store(c, b).start()
        return _

    lax.fori_loop(0, n // NBUF, outer, None)
    for c0 in range(n - NBUF + PD, n):
        store(c0, c0 % NBUF).wait()


def kernel(input_ids, table):
    R, S = input_ids.shape
    B = R * S
    mesh = plsc.VectorSubcoreMesh(
        core_axis_name="c", subcore_axis_name="s", num_cores=NC, num_subcores=NS
    )
    return pl.kernel(
        _body,
        out_type=jax.ShapeDtypeStruct((R, S, D_MODEL), jnp.float32),
        mesh=mesh,
        scratch_types=(
            [pltpu.VMEM((B // NW,), jnp.int32)]
            + [pltpu.VMEM((CHUNK, D_MODEL), jnp.float32)] * NBUF
            + [pltpu.SemaphoreType.DMA] * NBUF
            + [pltpu.SemaphoreType.DMA] * NBUF
        ),
    )(input_ids, table)


# speedup vs baseline: 1.0111x; 1.0111x over previous
import functools
import jax
import jax.numpy as jnp
from jax import lax
from jax.experimental import pallas as pl
from jax.experimental.pallas import tpu as pltpu
from jax.experimental.pallas import tpu_sc as plsc

D_MODEL = 1024
SCALE = 32.0
NC, NS, L = 2, 16, 16
NW = NC * NS
CHUNK = 16
NBUF = 4
PD = 2


def _body(ids_hbm, table_hbm, out_hbm, *scratch):
    idx_v = scratch[0]
    bufs = scratch[1:1 + NBUF]
    gsems = scratch[1 + NBUF:1 + 2 * NBUF]
    ssems = scratch[1 + 2 * NBUF:1 + 3 * NBUF]
    R, S = ids_hbm.shape
    B = R * S
    bpw = B // NW
    n = bpw // CHUNK
    wid = lax.axis_index("s") * NC + lax.axis_index("c")
    base = wid * bpw
    r0 = base // S
    col0 = base % S

    pltpu.sync_copy(ids_hbm.at[r0, pl.ds(col0, bpw)], idx_v)

    def gather(c, b):
        return pltpu.make_async_copy(
            table_hbm.at[idx_v.at[pl.ds(c * CHUNK, CHUNK)]], bufs[b], gsems[b]
        )

    def store(c, b):
        return pltpu.make_async_copy(
            bufs[b], out_hbm.at[r0, pl.ds(col0 + c * CHUNK, CHUNK)], ssems[b]
        )

    for c0 in range(PD):
        gather(c0, c0).start()

    def scale_buf(buf):
        @plsc.parallel_loop(0, CHUNK, 1)
        def _(j):
            @plsc.parallel_loop(0, D_MODEL, L, unroll=8)
            def _(k):
                buf[j, pl.ds(k, L)] = buf[j, pl.ds(k, L)] * SCALE

    def outer(t, _):
        for b in range(NBUF):
            c = NBUF * t + b
            nb = (b + PD) % NBUF

            @pl.when(c >= NBUF - PD)
            def _():
                store(c + PD - NBUF, nb).wait()

            @pl.when(c + PD < n)
            def _():
                gather(c + PD, nb).start()

            gather(c, b).wait()
            scale_buf(bufs[b])
            store(c, b).start()
        return _

    lax.fori_loop(0, n // NBUF, outer, None)
    for c0 in range(n - NBUF + PD, n):
        store(c0, c0 % NBUF).wait()


def kernel(input_ids, table):
    R, S = input_ids.shape
    B = R * S
    mesh = plsc.VectorSubcoreMesh(
        core_axis_name="c", subcore_axis_name="s", num_cores=NC, num_subcores=NS
    )
    return pl.kernel(
        _body,
        out_type=jax.ShapeDtypeStruct((R, S, D_MODEL), jnp.float32),
        mesh=mesh,
        scratch_types=(
            [pltpu.VMEM((B // NW,), jnp.int32)]
            + [pltpu.VMEM((CHUNK, D_MODEL), jnp.float32)] * NBUF
            + [pltpu.SemaphoreType.DMA] * NBUF
            + [pltpu.SemaphoreType.DMA] * NBUF
        ),
    )(input_ids, table)
